# Initial kernel scaffold; baseline (speedup 1.0000x reference)
import jax
import jax.numpy as jnp
from jax.experimental import pallas as pl


def kernel(input, index):
    raise NotImplementedError("write your pallas kernel here")


# speedup vs baseline: 4.2564x; 4.2564x over previous
import jax
import jax.numpy as jnp
from jax import lax
from jax.experimental import pallas as pl
from jax.experimental.pallas import tpu as pltpu
from jax.experimental.pallas import tpu_sc as plsc

EPS = 1e-09
E = 320000
D = 128
N = 10000
NC = 2
NS = 16
NW = NC * NS
NPAD = 10240
SLAB = NPAD // NS
CW = 16
EPT = E // NW
R = 80
NCHUNK = EPT // R
NSLAB = SLAB // R


def _body_a(x_hbm, idx_hbm, zrow_hbm, zcnt_hbm, one_hbm,
            s0_hbm, s1_hbm, c0_hbm, c1_hbm,
            acc_sh, cnt_sh, zc_v, ones_v, idx_v, rows_v):
    c = lax.axis_index("c")
    s = lax.axis_index("s")
    row0 = s * SLAB
    pltpu.sync_copy(zrow_hbm, rows_v)
    pltpu.sync_copy(zcnt_hbm, zc_v)
    pltpu.sync_copy(one_hbm, ones_v)
    for j in range(NSLAB):
        pltpu.sync_copy(rows_v, acc_sh.at[pl.ds(row0 + j * R, R), :])
        pltpu.sync_copy(zc_v, cnt_sh.at[pl.ds(row0 + j * R, R), :])
    plsc.subcore_barrier()

    base = (c * NS + s) * EPT

    def step(i, carry):
        off = base + i * R
        pltpu.sync_copy(idx_hbm.at[pl.ds(off, R)], idx_v)
        pltpu.sync_copy(x_hbm.at[pl.ds(off, R), :], rows_v)
        pltpu.sync_copy(rows_v, acc_sh.at[idx_v], add=True)
        pltpu.sync_copy(ones_v, cnt_sh.at[idx_v], add=True)
        return carry

    lax.fori_loop(0, NCHUNK, step, 0)
    plsc.subcore_barrier()

    def dump(j, carry):
        r0 = row0 + j * R
        pltpu.sync_copy(acc_sh.at[pl.ds(r0, R), :], rows_v)
        pltpu.sync_copy(cnt_sh.at[pl.ds(r0, R), :], zc_v)

        @pl.when(c == 0)
        def _():
            pltpu.sync_copy(rows_v, s0_hbm.at[pl.ds(r0, R), :])
            pltpu.sync_copy(zc_v, c0_hbm.at[pl.ds(r0, R), :])

        @pl.when(c == 1)
        def _():
            pltpu.sync_copy(rows_v, s1_hbm.at[pl.ds(r0, R), :])
            pltpu.sync_copy(zc_v, c1_hbm.at[pl.ds(r0, R), :])

        return carry

    lax.fori_loop(0, NSLAB, dump, 0)


def _body_b(idx_hbm, s0_hbm, s1_hbm, c0_hbm, c1_hbm, out_hbm,
            pooled_sh, a_v, b_v, ca_v, cb_v, idx_v, rows_v):
    c = lax.axis_index("c")
    s = lax.axis_index("s")
    row0 = s * SLAB

    def comb(j, carry):
        r0 = row0 + j * R
        pltpu.sync_copy(s0_hbm.at[pl.ds(r0, R), :], a_v)
        pltpu.sync_copy(s1_hbm.at[pl.ds(r0, R), :], b_v)
        pltpu.sync_copy(c0_hbm.at[pl.ds(r0, R), :], ca_v)
        pltpu.sync_copy(c1_hbm.at[pl.ds(r0, R), :], cb_v)

        def nrow(r, cc):
            sv = ca_v[r, pl.ds(0, 16)] + cb_v[r, pl.ds(0, 16)] + jnp.float32(EPS)
            scale = jnp.float32(1.0) / sv
            for k in range(8):
                sl = pl.ds(k * 16, 16)
                a_v[r, sl] = (a_v[r, sl] + b_v[r, sl]) * scale
            return cc

        lax.fori_loop(0, R, nrow, 0)
        pltpu.sync_copy(a_v, pooled_sh.at[pl.ds(r0, R), :])
        return carry

    lax.fori_loop(0, NSLAB, comb, 0)
    plsc.subcore_barrier()

    base = (c * NS + s) * EPT

    def gstep(i, carry):
        off = base + i * R
        pltpu.sync_copy(idx_hbm.at[pl.ds(off, R)], idx_v)
        pltpu.sync_copy(pooled_sh.at[idx_v], rows_v)
        pltpu.sync_copy(rows_v, out_hbm.at[pl.ds(off, R), :])
        return carry

    lax.fori_loop(0, NCHUNK, gstep, 0)


def kernel(input, index):
    mesh = plsc.VectorSubcoreMesh(core_axis_name="c", subcore_axis_name="s",
                                  num_cores=NC, num_subcores=NS)
    f32 = jnp.float32
    zrow = jnp.zeros((R, D), f32)
    zcnt = jnp.zeros((R, CW), f32)
    ones = jnp.ones((R, CW), f32)

    cparams = pltpu.CompilerParams(use_tc_tiling_on_sc=False)
    ka = pl.kernel(
        _body_a,
        compiler_params=cparams,
        out_type=[jax.ShapeDtypeStruct((NPAD, D), f32),
                  jax.ShapeDtypeStruct((NPAD, D), f32),
                  jax.ShapeDtypeStruct((NPAD, CW), f32),
                  jax.ShapeDtypeStruct((NPAD, CW), f32)],
        mesh=mesh,
        scratch_types=[
            pltpu.VMEM_SHARED((NPAD, D), f32),
            pltpu.VMEM_SHARED((NPAD, CW), f32),
            pltpu.VMEM((R, CW), f32),
            pltpu.VMEM((R, CW), f32),
            pltpu.VMEM((R,), jnp.int32),
            pltpu.VMEM((R, D), f32),
        ],
    )
    s0, s1, c0, c1 = ka(input, index, zrow, zcnt, ones)

    kb = pl.kernel(
        _body_b,
        compiler_params=cparams,
        out_type=jax.ShapeDtypeStruct((E, D), f32),
        mesh=mesh,
        scratch_types=[
            pltpu.VMEM_SHARED((NPAD, D), f32),
            pltpu.VMEM((R, D), f32),
            pltpu.VMEM((R, D), f32),
            pltpu.VMEM((R, CW), f32),
            pltpu.VMEM((R, CW), f32),
            pltpu.VMEM((R,), jnp.int32),
            pltpu.VMEM((R, D), f32),
        ],
    )
    return kb(index, s0, s1, c0, c1)


# speedup vs baseline: 7.7540x; 1.8217x over previous
import jax
import jax.numpy as jnp
from jax import lax
from jax.experimental import pallas as pl
from jax.experimental.pallas import tpu as pltpu
from jax.experimental.pallas import tpu_sc as plsc

EPS = 1e-09
E = 320000
D = 128
N = 10000
NC = 2
NS = 16
NW = NC * NS
NPAD = 10240
SLAB = NPAD // NS
CW = 16
EPT = E // NW
R = 80
NCHUNK = EPT // R
NPAIR = (NCHUNK - 1) // 2
NSLAB = SLAB // R


def _body_a(x_hbm, idx_hbm, zrow_hbm, zcnt_hbm, one_hbm,
            s0_hbm, s1_hbm, c0_hbm, c1_hbm,
            acc_sh, cnt_sh, zc_v, ones_v,
            idx0_v, idx1_v, rows0_v, rows1_v,
            ld0_s, ld1_s, sc0_s, sc1_s):
    c = lax.axis_index("c")
    s = lax.axis_index("s")
    idxs = (idx0_v, idx1_v)
    rows = (rows0_v, rows1_v)
    lds = (ld0_s, ld1_s)
    scs = (sc0_s, sc1_s)
    row0 = s * SLAB
    pltpu.sync_copy(zrow_hbm, rows0_v)
    pltpu.sync_copy(zcnt_hbm, zc_v)
    pltpu.sync_copy(one_hbm, ones_v)
    for j in range(NSLAB):
        pltpu.sync_copy(rows0_v, acc_sh.at[pl.ds(row0 + j * R, R), :])
        pltpu.sync_copy(zc_v, cnt_sh.at[pl.ds(row0 + j * R, R), :])
    plsc.subcore_barrier()

    base = (c * NS + s) * EPT

    def start_load(off, b):
        pltpu.async_copy(idx_hbm.at[pl.ds(off, R)], idxs[b], lds[b])
        pltpu.async_copy(x_hbm.at[pl.ds(off, R), :], rows[b], lds[b])

    def wait_load(b):
        pltpu.make_async_copy(idx_hbm.at[pl.ds(0, R)], idxs[b], lds[b]).wait()
        pltpu.make_async_copy(x_hbm.at[pl.ds(0, R), :], rows[b], lds[b]).wait()

    start_load(base, 0)
    start_load(base + R, 1)

    def pair(i, carry):
        for b in range(2):
            ch = 2 * i + b
            wait_load(b)
            d1 = pltpu.async_copy(rows[b], acc_sh.at[idxs[b]], scs[b], add=True)
            d2 = pltpu.async_copy(ones_v, cnt_sh.at[idxs[b]], scs[b], add=True)
            d1.wait()
            d2.wait()

            @pl.when(ch + 2 < NCHUNK)
            def _():
                start_load(base + (ch + 2) * R, b)

        return carry

    lax.fori_loop(0, NPAIR, pair, 0)
    wait_load(0)
    pltpu.sync_copy(rows0_v, acc_sh.at[idx0_v], add=True)
    pltpu.sync_copy(ones_v, cnt_sh.at[idx0_v], add=True)
    plsc.subcore_barrier()

    def dump(j, carry):
        r0 = row0 + j * R
        pltpu.sync_copy(acc_sh.at[pl.ds(r0, R), :], rows0_v)
        pltpu.sync_copy(cnt_sh.at[pl.ds(r0, R), :], zc_v)

        @pl.when(c == 0)
        def _():
            pltpu.sync_copy(rows0_v, s0_hbm.at[pl.ds(r0, R), :])
            pltpu.sync_copy(zc_v, c0_hbm.at[pl.ds(r0, R), :])

        @pl.when(c == 1)
        def _():
            pltpu.sync_copy(rows0_v, s1_hbm.at[pl.ds(r0, R), :])
            pltpu.sync_copy(zc_v, c1_hbm.at[pl.ds(r0, R), :])

        return carry

    lax.fori_loop(0, NSLAB, dump, 0)


def _body_b(idx_hbm, s0_hbm, s1_hbm, c0_hbm, c1_hbm, out_hbm,
            pooled_sh, ca_v, cb_v,
            idx0_v, idx1_v, rows0_v, rows1_v,
            ld0_s, ld1_s, g_s, st0_s, st1_s):
    c = lax.axis_index("c")
    s = lax.axis_index("s")
    idxs = (idx0_v, idx1_v)
    rows = (rows0_v, rows1_v)
    lds = (ld0_s, ld1_s)
    sts = (st0_s, st1_s)
    row0 = s * SLAB

    def comb(j, carry):
        r0 = row0 + j * R
        pltpu.sync_copy(s0_hbm.at[pl.ds(r0, R), :], rows0_v)
        pltpu.sync_copy(s1_hbm.at[pl.ds(r0, R), :], rows1_v)
        pltpu.sync_copy(c0_hbm.at[pl.ds(r0, R), :], ca_v)
        pltpu.sync_copy(c1_hbm.at[pl.ds(r0, R), :], cb_v)

        def nrow(r, cc):
            sv = ca_v[r, pl.ds(0, 16)] + cb_v[r, pl.ds(0, 16)] + jnp.float32(EPS)
            scale = jnp.float32(1.0) / sv
            for k in range(8):
                sl = pl.ds(k * 16, 16)
                rows0_v[r, sl] = (rows0_v[r, sl] + rows1_v[r, sl]) * scale
            return cc

        lax.fori_loop(0, R, nrow, 0)
        pltpu.sync_copy(rows0_v, pooled_sh.at[pl.ds(r0, R), :])
        return carry

    lax.fori_loop(0, NSLAB, comb, 0)
    plsc.subcore_barrier()

    base = (c * NS + s) * EPT

    def wait_idx(b):
        pltpu.make_async_copy(idx_hbm.at[pl.ds(0, R)], idxs[b], lds[b]).wait()

    def wait_store(b):
        pltpu.make_async_copy(rows[b], out_hbm.at[pl.ds(0, R), :], sts[b]).wait()

    pltpu.async_copy(idx_hbm.at[pl.ds(base, R)], idx0_v, ld0_s)
    pltpu.async_copy(idx_hbm.at[pl.ds(base + R, R)], idx1_v, ld1_s)

    def gpair(i, carry):
        for b in range(2):
            ch = 2 * i + b
            wait_idx(b)

            @pl.when(ch >= 2)
            def _():
                wait_store(b)

            g = pltpu.async_copy(pooled_sh.at[idxs[b]], rows[b], g_s)
            g.wait()
            pltpu.async_copy(rows[b], out_hbm.at[pl.ds(base + ch * R, R), :],
                             sts[b])

            @pl.when(ch + 2 < NCHUNK)
            def _():
                pltpu.async_copy(idx_hbm.at[pl.ds(base + (ch + 2) * R, R)],
                                 idxs[b], lds[b])

        return carry

    lax.fori_loop(0, NPAIR, gpair, 0)
    wait_idx(0)
    wait_store(0)
    pltpu.sync_copy(pooled_sh.at[idx0_v], rows0_v)
    pltpu.sync_copy(rows0_v, out_hbm.at[pl.ds(base + (NCHUNK - 1) * R, R), :])
    wait_store(1)


def kernel(input, index):
    mesh = plsc.VectorSubcoreMesh(core_axis_name="c", subcore_axis_name="s",
                                  num_cores=NC, num_subcores=NS)
    f32 = jnp.float32
    zrow = jnp.zeros((R, D), f32)
    zcnt = jnp.zeros((R, CW), f32)
    ones = jnp.ones((R, CW), f32)

    cparams = pltpu.CompilerParams(use_tc_tiling_on_sc=False)
    ka = pl.kernel(
        _body_a,
        compiler_params=cparams,
        out_type=[jax.ShapeDtypeStruct((NPAD, D), f32),
                  jax.ShapeDtypeStruct((NPAD, D), f32),
                  jax.ShapeDtypeStruct((NPAD, CW), f32),
                  jax.ShapeDtypeStruct((NPAD, CW), f32)],
        mesh=mesh,
        scratch_types=[
            pltpu.VMEM_SHARED((NPAD, D), f32),
            pltpu.VMEM_SHARED((NPAD, CW), f32),
            pltpu.VMEM((R, CW), f32),
            pltpu.VMEM((R, CW), f32),
            pltpu.VMEM((R,), jnp.int32),
            pltpu.VMEM((R,), jnp.int32),
            pltpu.VMEM((R, D), f32),
            pltpu.VMEM((R, D), f32),
            pltpu.SemaphoreType.DMA,
            pltpu.SemaphoreType.DMA,
            pltpu.SemaphoreType.DMA,
            pltpu.SemaphoreType.DMA,
        ],
    )
    s0, s1, c0, c1 = ka(input, index, zrow, zcnt, ones)

    kb = pl.kernel(
        _body_b,
        compiler_params=cparams,
        out_type=jax.ShapeDtypeStruct((E, D), f32),
        mesh=mesh,
        scratch_types=[
            pltpu.VMEM_SHARED((NPAD, D), f32),
            pltpu.VMEM((R, CW), f32),
            pltpu.VMEM((R, CW), f32),
            pltpu.VMEM((R,), jnp.int32),
            pltpu.VMEM((R,), jnp.int32),
            pltpu.VMEM((R, D), f32),
            pltpu.VMEM((R, D), f32),
            pltpu.SemaphoreType.DMA,
            pltpu.SemaphoreType.DMA,
            pltpu.SemaphoreType.DMA,
            pltpu.SemaphoreType.DMA,
            pltpu.SemaphoreType.DMA,
        ],
    )
    return kb(index, s0, s1, c0, c1)


# speedup vs baseline: 8.4719x; 1.0926x over previous
import jax
import jax.numpy as jnp
from jax import lax
from jax.experimental import pallas as pl
from jax.experimental.pallas import tpu as pltpu
from jax.experimental.pallas import tpu_sc as plsc

EPS = 1e-09
E = 320000
D = 128
N = 10000
NC = 2
NS = 16
NW = NC * NS
NPAD = 10240
SLAB = NPAD // NS
CW = 16
EPT = E // NW
R = 128
NFULL = EPT // R
TR = EPT - NFULL * R
NPAIR = NFULL // 2
NSLAB = SLAB // R


def _body_a(x_hbm, idx_hbm, zrow_hbm, zcnt_hbm, one_hbm,
            s0_hbm, s1_hbm, c0_hbm, c1_hbm,
            acc_sh, cnt_sh, zc_v, ones_v,
            idx0_v, idx1_v, idxt_v, rows0_v, rows1_v,
            ld0_s, ld1_s, sc0_s, sc1_s):
    c = lax.axis_index("c")
    s = lax.axis_index("s")
    idxs = (idx0_v, idx1_v)
    rows = (rows0_v, rows1_v)
    lds = (ld0_s, ld1_s)
    scs = (sc0_s, sc1_s)
    row0 = s * SLAB
    pltpu.sync_copy(zrow_hbm, rows0_v)
    pltpu.sync_copy(zcnt_hbm, zc_v)
    pltpu.sync_copy(one_hbm, ones_v)
    for j in range(NSLAB):
        pltpu.sync_copy(rows0_v, acc_sh.at[pl.ds(row0 + j * R, R), :])
        pltpu.sync_copy(zc_v, cnt_sh.at[pl.ds(row0 + j * R, R), :])
    plsc.subcore_barrier()

    base = (c * NS + s) * EPT

    def start_load(off, b):
        pltpu.async_copy(idx_hbm.at[pl.ds(off, R)], idxs[b], lds[b])
        pltpu.async_copy(x_hbm.at[pl.ds(off, R), :], rows[b], lds[b])

    def wait_load(b):
        pltpu.make_async_copy(idx_hbm.at[pl.ds(0, R)], idxs[b], lds[b]).wait()
        pltpu.make_async_copy(x_hbm.at[pl.ds(0, R), :], rows[b], lds[b]).wait()

    start_load(base, 0)
    start_load(base + R, 1)

    def pair(i, carry):
        for b in range(2):
            ch = 2 * i + b
            wait_load(b)
            d1 = pltpu.async_copy(rows[b], acc_sh.at[idxs[b]], scs[b], add=True)
            d2 = pltpu.async_copy(ones_v, cnt_sh.at[idxs[b]], scs[b], add=True)
            d1.wait()
            d2.wait()

            @pl.when(ch + 2 < NFULL)
            def _():
                start_load(base + (ch + 2) * R, b)

        return carry

    lax.fori_loop(0, NPAIR, pair, 0)
    pltpu.sync_copy(idx_hbm.at[pl.ds(base + NFULL * R, TR)], idxt_v)
    pltpu.sync_copy(x_hbm.at[pl.ds(base + NFULL * R, TR), :],
                    rows0_v.at[pl.ds(0, TR), :])
    pltpu.sync_copy(rows0_v.at[pl.ds(0, TR), :], acc_sh.at[idxt_v], add=True)
    pltpu.sync_copy(ones_v.at[pl.ds(0, TR), :], cnt_sh.at[idxt_v], add=True)
    plsc.subcore_barrier()

    def dump(j, carry):
        r0 = row0 + j * R
        pltpu.sync_copy(acc_sh.at[pl.ds(r0, R), :], rows0_v)
        pltpu.sync_copy(cnt_sh.at[pl.ds(r0, R), :], zc_v)

        @pl.when(c == 0)
        def _():
            pltpu.sync_copy(rows0_v, s0_hbm.at[pl.ds(r0, R), :])
            pltpu.sync_copy(zc_v, c0_hbm.at[pl.ds(r0, R), :])

        @pl.when(c == 1)
        def _():
            pltpu.sync_copy(rows0_v, s1_hbm.at[pl.ds(r0, R), :])
            pltpu.sync_copy(zc_v, c1_hbm.at[pl.ds(r0, R), :])

        return carry

    lax.fori_loop(0, NSLAB, dump, 0)


def _body_b(idx_hbm, s0_hbm, s1_hbm, c0_hbm, c1_hbm, out_hbm,
            pooled_sh, ca_v, cb_v,
            idx0_v, idx1_v, idxt_v, rows0_v, rows1_v, rowst_v,
            ld0_s, ld1_s, g_s, st0_s, st1_s):
    c = lax.axis_index("c")
    s = lax.axis_index("s")
    idxs = (idx0_v, idx1_v)
    rows = (rows0_v, rows1_v)
    lds = (ld0_s, ld1_s)
    sts = (st0_s, st1_s)
    row0 = s * SLAB

    def comb(j, carry):
        r0 = row0 + j * R
        pltpu.async_copy(s0_hbm.at[pl.ds(r0, R), :], rows0_v, g_s)
        pltpu.async_copy(s1_hbm.at[pl.ds(r0, R), :], rows1_v, g_s)
        pltpu.async_copy(c0_hbm.at[pl.ds(r0, R), :], ca_v, g_s)
        pltpu.async_copy(c1_hbm.at[pl.ds(r0, R), :], cb_v, g_s)
        pltpu.make_async_copy(s0_hbm.at[pl.ds(0, R), :], rows0_v, g_s).wait()
        pltpu.make_async_copy(s1_hbm.at[pl.ds(0, R), :], rows1_v, g_s).wait()
        pltpu.make_async_copy(c0_hbm.at[pl.ds(0, R), :], ca_v, g_s).wait()
        pltpu.make_async_copy(c1_hbm.at[pl.ds(0, R), :], cb_v, g_s).wait()

        def nrow(r, cc):
            sv = ca_v[r, pl.ds(0, 16)] + cb_v[r, pl.ds(0, 16)] + jnp.float32(EPS)
            scale = jnp.float32(1.0) / sv
            for k in range(8):
                sl = pl.ds(k * 16, 16)
                rows0_v[r, sl] = (rows0_v[r, sl] + rows1_v[r, sl]) * scale
            return cc

        lax.fori_loop(0, R, nrow, 0)
        pltpu.sync_copy(rows0_v, pooled_sh.at[pl.ds(r0, R), :])
        return carry

    lax.fori_loop(0, NSLAB, comb, 0)
    plsc.subcore_barrier()

    base = (c * NS + s) * EPT

    def wait_idx(b):
        pltpu.make_async_copy(idx_hbm.at[pl.ds(0, R)], idxs[b], lds[b]).wait()

    def wait_store(b):
        pltpu.make_async_copy(rows[b], out_hbm.at[pl.ds(0, R), :], sts[b]).wait()

    pltpu.async_copy(idx_hbm.at[pl.ds(base, R)], idx0_v, ld0_s)
    pltpu.async_copy(idx_hbm.at[pl.ds(base + R, R)], idx1_v, ld1_s)

    def gpair(i, carry):
        for b in range(2):
            ch = 2 * i + b
            wait_idx(b)

            @pl.when(ch >= 2)
            def _():
                wait_store(b)

            g = pltpu.async_copy(pooled_sh.at[idxs[b]], rows[b], g_s)
            g.wait()
            pltpu.async_copy(rows[b], out_hbm.at[pl.ds(base + ch * R, R), :],
                             sts[b])

            @pl.when(ch + 2 < NFULL)
            def _():
                pltpu.async_copy(idx_hbm.at[pl.ds(base + (ch + 2) * R, R)],
                                 idxs[b], lds[b])

        return carry

    lax.fori_loop(0, NPAIR, gpair, 0)
    pltpu.sync_copy(idx_hbm.at[pl.ds(base + NFULL * R, TR)], idxt_v)
    pltpu.sync_copy(pooled_sh.at[idxt_v], rowst_v)
    pltpu.sync_copy(rowst_v, out_hbm.at[pl.ds(base + NFULL * R, TR), :])
    wait_store(0)
    wait_store(1)


def kernel(input, index):
    mesh = plsc.VectorSubcoreMesh(core_axis_name="c", subcore_axis_name="s",
                                  num_cores=NC, num_subcores=NS)
    f32 = jnp.float32
    zrow = jnp.zeros((R, D), f32)
    zcnt = jnp.zeros((R, CW), f32)
    ones = jnp.ones((R, CW), f32)

    cparams = pltpu.CompilerParams(use_tc_tiling_on_sc=False)
    ka = pl.kernel(
        _body_a,
        compiler_params=cparams,
        out_type=[jax.ShapeDtypeStruct((NPAD, D), f32),
                  jax.ShapeDtypeStruct((NPAD, D), f32),
                  jax.ShapeDtypeStruct((NPAD, CW), f32),
                  jax.ShapeDtypeStruct((NPAD, CW), f32)],
        mesh=mesh,
        scratch_types=[
            pltpu.VMEM_SHARED((NPAD, D), f32),
            pltpu.VMEM_SHARED((NPAD, CW), f32),
            pltpu.VMEM((R, CW), f32),
            pltpu.VMEM((R, CW), f32),
            pltpu.VMEM((R,), jnp.int32),
            pltpu.VMEM((R,), jnp.int32),
            pltpu.VMEM((TR,), jnp.int32),
            pltpu.VMEM((R, D), f32),
            pltpu.VMEM((R, D), f32),
            pltpu.SemaphoreType.DMA,
            pltpu.SemaphoreType.DMA,
            pltpu.SemaphoreType.DMA,
            pltpu.SemaphoreType.DMA,
        ],
    )
    s0, s1, c0, c1 = ka(input, index, zrow, zcnt, ones)

    kb = pl.kernel(
        _body_b,
        compiler_params=cparams,
        out_type=jax.ShapeDtypeStruct((E, D), f32),
        mesh=mesh,
        scratch_types=[
            pltpu.VMEM_SHARED((NPAD, D), f32),
            pltpu.VMEM((R, CW), f32),
            pltpu.VMEM((R, CW), f32),
            pltpu.VMEM((R,), jnp.int32),
            pltpu.VMEM((R,), jnp.int32),
            pltpu.VMEM((TR,), jnp.int32),
            pltpu.VMEM((R, D), f32),
            pltpu.VMEM((R, D), f32),
            pltpu.VMEM((TR, D), f32),
            pltpu.SemaphoreType.DMA,
            pltpu.SemaphoreType.DMA,
            pltpu.SemaphoreType.DMA,
            pltpu.SemaphoreType.DMA,
            pltpu.SemaphoreType.DMA,
        ],
    )
    return kb(index, s0, s1, c0, c1)
